# baseline (speedup 1.0000x reference)
import jax
import jax.numpy as jnp
from jax.experimental import pallas as pl
from jax.experimental.pallas import tpu as pltpu

NUM_FRAMES = 4096
NUM_CLASSES = 1024
BLOCK_ROWS = 256
NUM_BLOCKS = NUM_FRAMES // BLOCK_ROWS
BLANK = 0
NEG = -2147483648


def _decode_block(x_ref, out_ref, carry_ref):
    i = pl.program_id(0)

    @pl.when(i == 0)
    def _init():
        carry_ref[0] = jnp.int32(-1)

    x = x_ref[...]
    m = jnp.max(x, axis=1, keepdims=True)
    cls = jax.lax.broadcasted_iota(jnp.int32, x.shape, 1)
    idx = jnp.min(jnp.where(x == m, cls, NUM_CLASSES), axis=1)
    idx = idx.reshape(1, BLOCK_ROWS)

    carry = carry_ref[0]
    pos = jax.lax.broadcasted_iota(jnp.int32, (1, BLOCK_ROWS), 1)
    prev = jnp.where(pos == 0, carry, jnp.roll(idx, 1, axis=1))
    keep = (idx != prev) & (idx != BLANK)
    out_ref[...] = jnp.where(keep, idx, jnp.int32(-1)).reshape(1, 1, BLOCK_ROWS)

    carry_ref[0] = jnp.max(jnp.where(pos == BLOCK_ROWS - 1, idx, NEG))


def kernel(emission):
    out = pl.pallas_call(
        _decode_block,
        grid=(NUM_BLOCKS,),
        in_specs=[
            pl.BlockSpec((BLOCK_ROWS, NUM_CLASSES), lambda i: (i, 0)),
        ],
        out_specs=pl.BlockSpec((1, 1, BLOCK_ROWS), lambda i: (i, 0, 0)),
        out_shape=jax.ShapeDtypeStruct((NUM_BLOCKS, 1, BLOCK_ROWS), jnp.int32),
        scratch_shapes=[pltpu.SMEM((1,), jnp.int32)],
    )(emission)
    return out.reshape(NUM_FRAMES)


# speedup vs baseline: 1.6697x; 1.6697x over previous
import jax
import jax.numpy as jnp
from jax.experimental import pallas as pl
from jax.experimental.pallas import tpu as pltpu

NUM_FRAMES = 4096
NUM_CLASSES = 1024
BLOCK_ROWS = 1024
NUM_BLOCKS = NUM_FRAMES // BLOCK_ROWS
BLANK = 0
NEG = -2147483648


def _decode_block(x_ref, out_ref, carry_ref):
    i = pl.program_id(0)

    @pl.when(i == 0)
    def _init():
        carry_ref[0] = jnp.int32(-1)

    x = x_ref[...]
    m = jnp.max(x, axis=1, keepdims=True)
    cls = jax.lax.broadcasted_iota(jnp.int32, x.shape, 1)
    idx = jnp.min(jnp.where(x == m, cls, NUM_CLASSES), axis=1)
    idx = idx.reshape(1, BLOCK_ROWS)

    carry = carry_ref[0]
    pos = jax.lax.broadcasted_iota(jnp.int32, (1, BLOCK_ROWS), 1)
    prev = jnp.where(pos == 0, carry, jnp.roll(idx, 1, axis=1))
    keep = (idx != prev) & (idx != BLANK)
    out_ref[...] = jnp.where(keep, idx, jnp.int32(-1)).reshape(1, 1, BLOCK_ROWS)

    carry_ref[0] = jnp.max(jnp.where(pos == BLOCK_ROWS - 1, idx, NEG))


def kernel(emission):
    out = pl.pallas_call(
        _decode_block,
        grid=(NUM_BLOCKS,),
        in_specs=[
            pl.BlockSpec((BLOCK_ROWS, NUM_CLASSES), lambda i: (i, 0)),
        ],
        out_specs=pl.BlockSpec((1, 1, BLOCK_ROWS), lambda i: (i, 0, 0)),
        out_shape=jax.ShapeDtypeStruct((NUM_BLOCKS, 1, BLOCK_ROWS), jnp.int32),
        scratch_shapes=[pltpu.SMEM((1,), jnp.int32)],
    )(emission)
    return out.reshape(NUM_FRAMES)


# speedup vs baseline: 1.6717x; 1.0012x over previous
import jax
import jax.numpy as jnp
from jax.experimental import pallas as pl
from jax.experimental.pallas import tpu as pltpu

NUM_FRAMES = 4096
NUM_CLASSES = 1024
BLOCK_ROWS = 2048
NUM_BLOCKS = NUM_FRAMES // BLOCK_ROWS
BLANK = 0
NEG = -2147483648


def _decode_block(x_ref, out_ref, carry_ref):
    i = pl.program_id(0)

    @pl.when(i == 0)
    def _init():
        carry_ref[0] = jnp.int32(-1)

    x = x_ref[...]
    m = jnp.max(x, axis=1, keepdims=True)
    cls = jax.lax.broadcasted_iota(jnp.int32, x.shape, 1)
    idx = jnp.min(jnp.where(x == m, cls, NUM_CLASSES), axis=1)
    idx = idx.reshape(1, BLOCK_ROWS)

    carry = carry_ref[0]
    pos = jax.lax.broadcasted_iota(jnp.int32, (1, BLOCK_ROWS), 1)
    prev = jnp.where(pos == 0, carry, jnp.roll(idx, 1, axis=1))
    keep = (idx != prev) & (idx != BLANK)
    out_ref[...] = jnp.where(keep, idx, jnp.int32(-1)).reshape(1, 1, BLOCK_ROWS)

    carry_ref[0] = jnp.max(jnp.where(pos == BLOCK_ROWS - 1, idx, NEG))


def kernel(emission):
    out = pl.pallas_call(
        _decode_block,
        grid=(NUM_BLOCKS,),
        in_specs=[
            pl.BlockSpec((BLOCK_ROWS, NUM_CLASSES), lambda i: (i, 0)),
        ],
        out_specs=pl.BlockSpec((1, 1, BLOCK_ROWS), lambda i: (i, 0, 0)),
        out_shape=jax.ShapeDtypeStruct((NUM_BLOCKS, 1, BLOCK_ROWS), jnp.int32),
        scratch_shapes=[pltpu.SMEM((1,), jnp.int32)],
    )(emission)
    return out.reshape(NUM_FRAMES)
